# baseline (speedup 1.0000x reference)
import functools

import jax
import jax.numpy as jnp
from jax import lax
from jax.experimental import pallas as pl
from jax.experimental.pallas import tpu as pltpu

N = 4096
NFEAT = 512
NHID = 256
NCLASS = 40
NCPAD = 128
BM = 512
NBLK = N // BM


def _gcn_kernel(x_ref, w1_ref, w2_ref, adj_ref, logp_ref, h_ref,
                adjb_s, xw1_s, hw2_s):
    i = pl.program_id(0)

    @pl.when(i < NBLK)
    def _phase_a():
        xw1_s[pl.ds(i * BM, BM), :] = jnp.dot(
            x_ref[...], w1_ref[...],
            preferred_element_type=jnp.float32).astype(jnp.bfloat16)

    @pl.when(jnp.logical_and(i >= NBLK, i < 2 * NBLK))
    def _phase_b():
        k = i - NBLK
        rk = pl.ds(k * BM, BM)
        adjb_s[rk, :] = adj_ref[...].astype(jnp.bfloat16)
        hb = jnp.maximum(
            jnp.dot(adjb_s[rk, :], xw1_s[...],
                    preferred_element_type=jnp.float32),
            0.0)
        h_ref[...] = hb
        hw2_s[rk, :] = jnp.dot(
            hb.astype(jnp.bfloat16), w2_ref[...],
            preferred_element_type=jnp.float32).astype(jnp.bfloat16)
        logp_ref[rk, :] = jnp.zeros((BM, NCLASS), jnp.float32)

        def _row_k_vs_old_cols(c, carry):
            logp_ref[rk, :] += jnp.dot(
                adjb_s[rk, pl.ds(c * BM, BM)], hw2_s[pl.ds(c * BM, BM), :],
                preferred_element_type=jnp.float32)[:, :NCLASS]
            return carry

        lax.fori_loop(0, k, _row_k_vs_old_cols, 0, unroll=False)

        def _all_rows_vs_col_k(j, carry):
            rj = pl.ds(j * BM, BM)
            logp_ref[rj, :] += jnp.dot(
                adjb_s[rj, pl.ds(k * BM, BM)], hw2_s[rk, :],
                preferred_element_type=jnp.float32)[:, :NCLASS]
            return carry

        lax.fori_loop(0, k + 1, _all_rows_vs_col_k, 0, unroll=False)

    @pl.when(i == 2 * NBLK)
    def _phase_c():
        def _softmax_block(j, carry):
            rj = pl.ds(j * BM, BM)
            zr = jnp.maximum(logp_ref[rj, :], 0.0)
            m = jnp.max(zr, axis=1, keepdims=True)
            s = jnp.sum(jnp.exp(zr - m), axis=1, keepdims=True)
            logp_ref[rj, :] = zr - m - jnp.log(s)
            return carry

        lax.fori_loop(0, NBLK, _softmax_block, 0, unroll=False)


@functools.partial(jax.jit, static_argnames=())
def kernel(x, adj, W1, W2):
    w2p = jnp.pad(W2, ((0, 0), (0, NCPAD - NCLASS))).astype(jnp.bfloat16)
    grid = (2 * NBLK + 1,)
    logp, h = pl.pallas_call(
        _gcn_kernel,
        grid=grid,
        in_specs=[
            pl.BlockSpec((BM, NFEAT), lambda i: (jnp.minimum(i, NBLK - 1), 0)),
            pl.BlockSpec((NFEAT, NHID), lambda i: (0, 0)),
            pl.BlockSpec((NHID, NCPAD), lambda i: (0, 0)),
            pl.BlockSpec((BM, N), lambda i: (jnp.clip(i - NBLK, 0, NBLK - 1), 0)),
        ],
        out_specs=[
            pl.BlockSpec((N, NCLASS), lambda i: (0, 0)),
            pl.BlockSpec((BM, NHID),
                         lambda i: (jnp.clip(i - NBLK, 0, NBLK - 1), 0)),
        ],
        out_shape=[
            jax.ShapeDtypeStruct((N, NCLASS), jnp.float32),
            jax.ShapeDtypeStruct((N, NHID), jnp.float32),
        ],
        scratch_shapes=[
            pltpu.VMEM((N, N), jnp.bfloat16),
            pltpu.VMEM((N, NHID), jnp.bfloat16),
            pltpu.VMEM((N, NCPAD), jnp.bfloat16),
        ],
        compiler_params=pltpu.CompilerParams(
            dimension_semantics=("arbitrary",),
        ),
    )(x, W1, w2p, adj)
    return (logp, h)


# speedup vs baseline: 1.0409x; 1.0409x over previous
import functools

import jax
import jax.numpy as jnp
from jax import lax
from jax.experimental import pallas as pl
from jax.experimental.pallas import tpu as pltpu

N = 4096
NFEAT = 512
NHID = 256
NCLASS = 40
NCPAD = 128
BM = 512
NBLK = N // BM


def _gcn_kernel(x_ref, w1_ref, w2_ref, adj_ref, logp_ref, h_ref,
                adjb_s, xw1_s, hw2_s, z_s):
    i = pl.program_id(0)

    @pl.when(i < NBLK)
    def _phase_a():
        xw1_s[pl.ds(i * BM, BM), :] = jnp.dot(
            x_ref[...], w1_ref[...],
            preferred_element_type=jnp.float32).astype(jnp.bfloat16)

    @pl.when(jnp.logical_and(i >= NBLK, i < 2 * NBLK))
    def _phase_b():
        k = i - NBLK
        rk = pl.ds(k * BM, BM)
        adjb_s[rk, :] = adj_ref[...].astype(jnp.bfloat16)
        hb = jnp.maximum(
            jnp.dot(adjb_s[rk, :], xw1_s[...],
                    preferred_element_type=jnp.float32),
            0.0)
        h_ref[...] = hb
        hw2_s[rk, :] = jnp.dot(
            hb.astype(jnp.bfloat16), w2_ref[...],
            preferred_element_type=jnp.float32).astype(jnp.bfloat16)
        acc = jnp.zeros((BM, NCPAD), jnp.float32)

        def _row_k_vs_old_cols(c, acc):
            return acc + jnp.dot(
                adjb_s[rk, pl.ds(c * BM, BM)], hw2_s[pl.ds(c * BM, BM), :],
                preferred_element_type=jnp.float32)

        acc = lax.fori_loop(0, k, _row_k_vs_old_cols, acc, unroll=False)
        z_s[rk, :] = acc.astype(jnp.bfloat16)

        zfull = z_s[...].astype(jnp.float32) + jnp.dot(
            adjb_s[:, rk], hw2_s[rk, :],
            preferred_element_type=jnp.float32)
        z_s[...] = zfull.astype(jnp.bfloat16)

    @pl.when(i == 2 * NBLK)
    def _phase_c():
        def _softmax_block(j, carry):
            rj = pl.ds(j * BM, BM)
            zr = jnp.maximum(z_s[rj, :].astype(jnp.float32), 0.0)
            col = lax.broadcasted_iota(jnp.int32, (BM, NCPAD), 1)
            valid = col < NCLASS
            zm = jnp.where(valid, zr, -jnp.inf)
            m = jnp.max(zm, axis=1, keepdims=True)
            s = jnp.sum(jnp.where(valid, jnp.exp(zm - m), 0.0),
                        axis=1, keepdims=True)
            logp_ref[rj, :] = (zr - m - jnp.log(s))[:, :NCLASS]
            return carry

        lax.fori_loop(0, NBLK, _softmax_block, 0, unroll=False)


@functools.partial(jax.jit, static_argnames=())
def kernel(x, adj, W1, W2):
    w2p = jnp.pad(W2, ((0, 0), (0, NCPAD - NCLASS))).astype(jnp.bfloat16)
    grid = (2 * NBLK + 1,)
    logp, h = pl.pallas_call(
        _gcn_kernel,
        grid=grid,
        in_specs=[
            pl.BlockSpec((BM, NFEAT), lambda i: (jnp.minimum(i, NBLK - 1), 0)),
            pl.BlockSpec((NFEAT, NHID), lambda i: (0, 0)),
            pl.BlockSpec((NHID, NCPAD), lambda i: (0, 0)),
            pl.BlockSpec((BM, N), lambda i: (jnp.clip(i - NBLK, 0, NBLK - 1), 0)),
        ],
        out_specs=[
            pl.BlockSpec((N, NCLASS), lambda i: (0, 0)),
            pl.BlockSpec((BM, NHID),
                         lambda i: (jnp.clip(i - NBLK, 0, NBLK - 1), 0)),
        ],
        out_shape=[
            jax.ShapeDtypeStruct((N, NCLASS), jnp.float32),
            jax.ShapeDtypeStruct((N, NHID), jnp.float32),
        ],
        scratch_shapes=[
            pltpu.VMEM((N, N), jnp.bfloat16),
            pltpu.VMEM((N, NHID), jnp.bfloat16),
            pltpu.VMEM((N, NCPAD), jnp.bfloat16),
            pltpu.VMEM((N, NCPAD), jnp.bfloat16),
        ],
        compiler_params=pltpu.CompilerParams(
            dimension_semantics=("arbitrary",),
        ),
    )(x, W1, w2p, adj)
    return (logp, h)


# speedup vs baseline: 1.0720x; 1.0299x over previous
import functools

import jax
import jax.numpy as jnp
from jax import lax
from jax.experimental import pallas as pl
from jax.experimental.pallas import tpu as pltpu

N = 4096
NFEAT = 512
NHID = 256
NCLASS = 40
NCPAD = 128
BM = 512
NBLK = N // BM


def _gcn_kernel(x_ref, w1_ref, w2_ref, adj_ref, logp_ref, h_ref,
                adjb_s, xw1_s, hw2_s):
    i = pl.program_id(0)

    @pl.when(i < NBLK)
    def _phase_a():
        xw1_s[pl.ds(i * BM, BM), :] = jnp.dot(
            x_ref[...], w1_ref[...],
            preferred_element_type=jnp.float32).astype(jnp.bfloat16)

    @pl.when(jnp.logical_and(i >= NBLK, i < 2 * NBLK))
    def _phase_b():
        k = i - NBLK
        adjb_s[pl.ds(k * BM, BM), :] = adj_ref[...].astype(jnp.bfloat16)
        hb = jnp.maximum(
            jnp.dot(adjb_s[pl.ds(k * BM, BM), :], xw1_s[...],
                    preferred_element_type=jnp.float32),
            0.0)
        h_ref[...] = hb
        hw2_s[pl.ds(k * BM, BM), :] = jnp.dot(
            hb.astype(jnp.bfloat16), w2_ref[...],
            preferred_element_type=jnp.float32).astype(jnp.bfloat16)

    @pl.when(i >= 2 * NBLK)
    def _phase_c():
        j = i - 2 * NBLK
        z = jnp.dot(adjb_s[pl.ds(j * BM, BM), :], hw2_s[...],
                    preferred_element_type=jnp.float32)
        zr = jnp.maximum(z, 0.0)
        col = lax.broadcasted_iota(jnp.int32, (BM, NCPAD), 1)
        valid = col < NCLASS
        zm = jnp.where(valid, zr, -jnp.inf)
        m = jnp.max(zm, axis=1, keepdims=True)
        s = jnp.sum(jnp.where(valid, jnp.exp(zm - m), 0.0),
                    axis=1, keepdims=True)
        logp_ref[...] = (zr - m - jnp.log(s))[:, :NCLASS]


@functools.partial(jax.jit, static_argnames=())
def kernel(x, adj, W1, W2):
    w2p = jnp.pad(W2, ((0, 0), (0, NCPAD - NCLASS))).astype(jnp.bfloat16)
    grid = (3 * NBLK,)
    logp, h = pl.pallas_call(
        _gcn_kernel,
        grid=grid,
        in_specs=[
            pl.BlockSpec((BM, NFEAT), lambda i: (jnp.minimum(i, NBLK - 1), 0)),
            pl.BlockSpec((NFEAT, NHID), lambda i: (0, 0)),
            pl.BlockSpec((NHID, NCPAD), lambda i: (0, 0)),
            pl.BlockSpec((BM, N), lambda i: (jnp.clip(i - NBLK, 0, NBLK - 1), 0)),
        ],
        out_specs=[
            pl.BlockSpec((BM, NCLASS),
                         lambda i: (jnp.maximum(i - 2 * NBLK, 0), 0)),
            pl.BlockSpec((BM, NHID),
                         lambda i: (jnp.clip(i - NBLK, 0, NBLK - 1), 0)),
        ],
        out_shape=[
            jax.ShapeDtypeStruct((N, NCLASS), jnp.float32),
            jax.ShapeDtypeStruct((N, NHID), jnp.float32),
        ],
        scratch_shapes=[
            pltpu.VMEM((N, N), jnp.bfloat16),
            pltpu.VMEM((N, NHID), jnp.bfloat16),
            pltpu.VMEM((N, NCPAD), jnp.bfloat16),
        ],
        compiler_params=pltpu.CompilerParams(
            dimension_semantics=("arbitrary",),
        ),
    )(x, W1, w2p, adj)
    return (logp, h)


# speedup vs baseline: 1.1236x; 1.0481x over previous
import functools

import jax
import jax.numpy as jnp
from jax import lax
from jax.experimental import pallas as pl
from jax.experimental.pallas import tpu as pltpu

N = 4096
NFEAT = 512
NHID = 256
NCLASS = 40
NCPAD = 128
BM = 512
NBLK = N // BM


def _gcn_kernel(x_ref, w1_ref, w2_ref, adj_ref, logp_ref, h_ref,
                adjb_s, xw1_s, hw2_s, w2p_s):
    i = pl.program_id(0)

    @pl.when(i == 0)
    def _pad_w2():
        w2p_s[:, :NCLASS] = w2_ref[...].astype(jnp.bfloat16)
        w2p_s[:, NCLASS:] = jnp.zeros((NHID, NCPAD - NCLASS), jnp.bfloat16)

    @pl.when(i < NBLK)
    def _phase_a():
        xw1_s[pl.ds(i * BM, BM), :] = jnp.dot(
            x_ref[...], w1_ref[...],
            preferred_element_type=jnp.float32).astype(jnp.bfloat16)

    @pl.when(jnp.logical_and(i >= NBLK, i < 2 * NBLK))
    def _phase_b():
        k = i - NBLK
        adjb_s[pl.ds(k * BM, BM), :] = adj_ref[...].astype(jnp.bfloat16)
        hb = jnp.maximum(
            jnp.dot(adjb_s[pl.ds(k * BM, BM), :], xw1_s[...],
                    preferred_element_type=jnp.float32),
            0.0)
        h_ref[...] = hb
        hw2_s[pl.ds(k * BM, BM), :] = jnp.dot(
            hb.astype(jnp.bfloat16), w2p_s[...],
            preferred_element_type=jnp.float32).astype(jnp.bfloat16)

    @pl.when(i >= 2 * NBLK)
    def _phase_c():
        j = i - 2 * NBLK
        z = jnp.dot(adjb_s[pl.ds(j * BM, BM), :], hw2_s[...],
                    preferred_element_type=jnp.float32)
        zr = jnp.maximum(z, 0.0)
        col = lax.broadcasted_iota(jnp.int32, (BM, NCPAD), 1)
        valid = col < NCLASS
        zm = jnp.where(valid, zr, -jnp.inf)
        m = jnp.max(zm, axis=1, keepdims=True)
        s = jnp.sum(jnp.where(valid, jnp.exp(zm - m), 0.0),
                    axis=1, keepdims=True)
        logp_ref[...] = (zr - m - jnp.log(s)).T[:NCLASS, :]


@functools.partial(jax.jit, static_argnames=())
def kernel(x, adj, W1, W2):
    grid = (3 * NBLK,)
    logp_t, h = pl.pallas_call(
        _gcn_kernel,
        grid=grid,
        in_specs=[
            pl.BlockSpec((BM, NFEAT), lambda i: (jnp.minimum(i, NBLK - 1), 0)),
            pl.BlockSpec((NFEAT, NHID), lambda i: (0, 0)),
            pl.BlockSpec((NHID, NCLASS), lambda i: (0, 0)),
            pl.BlockSpec((BM, N), lambda i: (jnp.clip(i - NBLK, 0, NBLK - 1), 0)),
        ],
        out_specs=[
            pl.BlockSpec((NCLASS, BM),
                         lambda i: (0, jnp.maximum(i - 2 * NBLK, 0))),
            pl.BlockSpec((BM, NHID),
                         lambda i: (jnp.clip(i - NBLK, 0, NBLK - 1), 0)),
        ],
        out_shape=[
            jax.ShapeDtypeStruct((NCLASS, N), jnp.float32),
            jax.ShapeDtypeStruct((N, NHID), jnp.float32),
        ],
        scratch_shapes=[
            pltpu.VMEM((N, N), jnp.bfloat16),
            pltpu.VMEM((N, NHID), jnp.bfloat16),
            pltpu.VMEM((N, NCPAD), jnp.bfloat16),
            pltpu.VMEM((NHID, NCPAD), jnp.bfloat16),
        ],
        compiler_params=pltpu.CompilerParams(
            dimension_semantics=("arbitrary",),
        ),
    )(x, W1, W2, adj)
    return (logp_t.T, h)


# speedup vs baseline: 1.1828x; 1.0528x over previous
import functools

import jax
import jax.numpy as jnp
from jax import lax
from jax.experimental import pallas as pl
from jax.experimental.pallas import tpu as pltpu

N = 4096
NFEAT = 512
NHID = 256
NCLASS = 40
NCPAD = 128
BM = 512
NBLK = N // BM


NHALF = N // 2
BZ = N // (NBLK // 2)


def _gcn_kernel(x_ref, w1_ref, w2_ref, adj_ref, logp_ref, h_ref,
                adjb_s, xw1_s, hw2_s, w2p_s, z_s):
    i = pl.program_id(0)

    @pl.when(i == 0)
    def _pad_w2():
        w2p_s[:, :NCLASS] = w2_ref[...].astype(jnp.bfloat16)
        w2p_s[:, NCLASS:] = jnp.zeros((NHID, NCPAD - NCLASS), jnp.bfloat16)

    @pl.when(i < NBLK)
    def _phase_a():
        xw1_s[pl.ds(i * BM, BM), :] = jnp.dot(
            x_ref[...], w1_ref[...],
            preferred_element_type=jnp.float32).astype(jnp.bfloat16)

    @pl.when(jnp.logical_and(i >= NBLK, i < 2 * NBLK))
    def _phase_b():
        k = i - NBLK
        adjb_s[pl.ds(k * BM, BM), :] = adj_ref[...].astype(jnp.bfloat16)
        hb = jnp.maximum(
            jnp.dot(adjb_s[pl.ds(k * BM, BM), :], xw1_s[...],
                    preferred_element_type=jnp.float32),
            0.0)
        h_ref[...] = hb
        hw2_s[pl.ds(k * BM, BM), :] = jnp.dot(
            hb.astype(jnp.bfloat16), w2p_s[...],
            preferred_element_type=jnp.float32).astype(jnp.bfloat16)

        @pl.when(k >= NBLK // 2)
        def _layer2_first_half_k():
            t = k - NBLK // 2
            rz = pl.ds(t * BZ, BZ)
            z_s[rz, :] = jnp.dot(
                adjb_s[rz, :NHALF], hw2_s[:NHALF, :],
                preferred_element_type=jnp.float32)

    @pl.when(i >= 2 * NBLK)
    def _phase_c():
        j = i - 2 * NBLK
        rj = pl.ds(j * BM, BM)
        z = z_s[rj, :] + jnp.dot(
            adjb_s[rj, NHALF:], hw2_s[NHALF:, :],
            preferred_element_type=jnp.float32)
        zr = jnp.maximum(z, 0.0)
        col = lax.broadcasted_iota(jnp.int32, (BM, NCPAD), 1)
        valid = col < NCLASS
        zm = jnp.where(valid, zr, -jnp.inf)
        m = jnp.max(zm, axis=1, keepdims=True)
        s = jnp.sum(jnp.where(valid, jnp.exp(zm - m), 0.0),
                    axis=1, keepdims=True)
        logp_ref[...] = (zr - m - jnp.log(s)).T[:NCLASS, :]


@functools.partial(jax.jit, static_argnames=())
def kernel(x, adj, W1, W2):
    grid = (3 * NBLK,)
    logp_t, h = pl.pallas_call(
        _gcn_kernel,
        grid=grid,
        in_specs=[
            pl.BlockSpec((BM, NFEAT), lambda i: (jnp.minimum(i, NBLK - 1), 0)),
            pl.BlockSpec((NFEAT, NHID), lambda i: (0, 0)),
            pl.BlockSpec((NHID, NCLASS), lambda i: (0, 0)),
            pl.BlockSpec((BM, N), lambda i: (jnp.clip(i - NBLK, 0, NBLK - 1), 0)),
        ],
        out_specs=[
            pl.BlockSpec((NCLASS, BM),
                         lambda i: (0, jnp.maximum(i - 2 * NBLK, 0))),
            pl.BlockSpec((BM, NHID),
                         lambda i: (jnp.clip(i - NBLK, 0, NBLK - 1), 0)),
        ],
        out_shape=[
            jax.ShapeDtypeStruct((NCLASS, N), jnp.float32),
            jax.ShapeDtypeStruct((N, NHID), jnp.float32),
        ],
        scratch_shapes=[
            pltpu.VMEM((N, N), jnp.bfloat16),
            pltpu.VMEM((N, NHID), jnp.bfloat16),
            pltpu.VMEM((N, NCPAD), jnp.bfloat16),
            pltpu.VMEM((NHID, NCPAD), jnp.bfloat16),
            pltpu.VMEM((N, NCPAD), jnp.float32),
        ],
        compiler_params=pltpu.CompilerParams(
            dimension_semantics=("arbitrary",),
        ),
    )(x, W1, W2, adj)
    return (logp_t.T, h)
